# Initial kernel scaffold; baseline (speedup 1.0000x reference)
import jax
import jax.numpy as jnp
from jax.experimental import pallas as pl


def kernel(x, edge_index, W1, a_src1, a_dst1, b1, g1, be1, rm1, rv1, W2, a_src2, a_dst2, b2, g2, be2, rm2, rv2, Wfc, bfc):
    raise NotImplementedError("write your pallas kernel here")


# speedup vs baseline: 38.2981x; 38.2981x over previous
import functools

import jax
import jax.numpy as jnp
from jax import lax
from jax.experimental import pallas as pl
from jax.experimental.pallas import tpu as pltpu
from jax.experimental.pallas import tpu_sc as plsc

N = 10000
E = 320000
D = 128
H = 4
C1 = 64
C2 = 32
EPS = 1e-5

NPAD = 10240
NR = NPAD // 128
KCH = 128
EPAD = 16 * KCH * 162
PER_TILE = EPAD // 16
NCH = PER_TILE // KCH
BN = 1024


def _k1_body(x_ref, w_ref, a_ref, ht_ref, aa_ref):
    h = jnp.dot(x_ref[...], w_ref[...], preferred_element_type=jnp.float32)
    for hh in range(H):
        ht_ref[:, hh, :] = h[:, hh * C1 : (hh + 1) * C1]
    aa_ref[...] = jnp.dot(h, a_ref[...], preferred_element_type=jnp.float32)


def _tc_layer1(xp, W1, A1):
    return pl.pallas_call(
        _k1_body,
        grid=(NPAD // BN,),
        in_specs=[
            pl.BlockSpec((BN, D), lambda i: (i, 0)),
            pl.BlockSpec((D, H * C1), lambda i: (0, 0)),
            pl.BlockSpec((H * C1, 2 * H), lambda i: (0, 0)),
        ],
        out_specs=[
            pl.BlockSpec((BN, H, C1), lambda i: (i, 0, 0)),
            pl.BlockSpec((BN, 2 * H), lambda i: (i, 0)),
        ],
        out_shape=[
            jax.ShapeDtypeStruct((NPAD, H, C1), jnp.float32),
            jax.ShapeDtypeStruct((NPAD, 2 * H), jnp.float32),
        ],
    )(xp, W1, A1)


def _k2_body(acc_ref, b_ref, g_ref, be_ref, rm_ref, rv_ref, w_ref, a_ref,
             ht_ref, aa_ref):
    h = jnp.concatenate(
        [acc_ref[0, 0], acc_ref[0, 1], acc_ref[1, 0], acc_ref[1, 1]],
        axis=-1) + b_ref[...]
    h = (h - rm_ref[...]) / jnp.sqrt(rv_ref[...] + EPS) * g_ref[...] + be_ref[...]
    z = jnp.where(h > 0, h, jnp.exp(h) - 1.0)
    h2 = jnp.dot(z, w_ref[...], preferred_element_type=jnp.float32)
    for hh in range(H):
        ht_ref[:, hh, :] = h2[:, hh * C2 : (hh + 1) * C2]
    aa_ref[...] = jnp.dot(h2, a_ref[...], preferred_element_type=jnp.float32)


def _tc_layer2(acc1, b1, g1, be1, rm1, rv1, W2, A2):
    vec = lambda: pl.BlockSpec((1, H * C1), lambda i: (0, 0))
    return pl.pallas_call(
        _k2_body,
        grid=(NPAD // BN,),
        in_specs=[
            pl.BlockSpec((2, 2, BN, C1), lambda i: (0, 0, i, 0)),
            vec(), vec(), vec(), vec(), vec(),
            pl.BlockSpec((H * C1, H * C2), lambda i: (0, 0)),
            pl.BlockSpec((H * C2, 2 * H), lambda i: (0, 0)),
        ],
        out_specs=[
            pl.BlockSpec((BN, H, C2), lambda i: (i, 0, 0)),
            pl.BlockSpec((BN, 2 * H), lambda i: (i, 0)),
        ],
        out_shape=[
            jax.ShapeDtypeStruct((NPAD, H, C2), jnp.float32),
            jax.ShapeDtypeStruct((NPAD, 2 * H), jnp.float32),
        ],
    )(acc1, b1, g1, be1, rm1, rv1, W2, A2)


def _k3_body(acc_ref, b_ref, g_ref, be_ref, rm_ref, rv_ref, w_ref, bfc_ref,
             out_ref):
    h = jnp.concatenate(
        [acc_ref[0, 0], acc_ref[0, 1], acc_ref[1, 0], acc_ref[1, 1]],
        axis=-1) + b_ref[...]
    h = (h - rm_ref[...]) / jnp.sqrt(rv_ref[...] + EPS) * g_ref[...] + be_ref[...]
    z = jnp.where(h > 0, h, jnp.exp(h) - 1.0)
    out_ref[...] = jnp.dot(z, w_ref[...], preferred_element_type=jnp.float32) + bfc_ref[...]


def _tc_final(acc2, b2, g2, be2, rm2, rv2, Wfc, bfc):
    vec = lambda: pl.BlockSpec((1, H * C2), lambda i: (0, 0))
    return pl.pallas_call(
        _k3_body,
        grid=(NPAD // BN,),
        in_specs=[
            pl.BlockSpec((2, 2, BN, C2), lambda i: (0, 0, i, 0)),
            vec(), vec(), vec(), vec(), vec(),
            pl.BlockSpec((H * C2, 1), lambda i: (0, 0)),
            pl.BlockSpec((1, 1), lambda i: (0, 0)),
        ],
        out_specs=pl.BlockSpec((BN, 1), lambda i: (i, 0)),
        out_shape=jax.ShapeDtypeStruct((NPAD, 1), jnp.float32),
    )(acc2, b2, g2, be2, rm2, rv2, Wfc, bfc)


_PASSB = True
_PASSA = True

def _make_sc_edge_kernel(C):
    mesh = plsc.VectorSubcoreMesh(core_axis_name="c", subcore_axis_name="s")
    scratch = [
        pltpu.VMEM((PER_TILE,), jnp.int32),
        pltpu.VMEM((PER_TILE,), jnp.int32),
        pltpu.VMEM((NPAD,), jnp.float32),
        pltpu.VMEM((NPAD,), jnp.float32),
        pltpu.VMEM((NR, 128), jnp.float32),
        pltpu.VMEM((KCH, C), jnp.float32),
        pltpu.VMEM((KCH,), jnp.int32),
        pltpu.VMEM((KCH,), jnp.int32),
        pltpu.VMEM((KCH,), jnp.float32),
        pltpu.VMEM((NR,), jnp.int32),
        pltpu.VMEM_SHARED((NPAD, C), jnp.float32),
        pltpu.VMEM_SHARED((NR, 128), jnp.float32),
        pltpu.SemaphoreType.DMA,
    ]
    NSL = NPAD // 16

    @functools.partial(
        pl.kernel,
        out_type=jax.ShapeDtypeStruct((2, 2, NPAD, C), jnp.float32),
        mesh=mesh,
        scratch_types=scratch,
        compiler_params=pltpu.CompilerParams(
            needs_layout_passes=False, use_tc_tiling_on_sc=False),
    )
    def k(ht, srcg, dstg, asad, out, srcb, dstb, tas, tad, den, rows,
          gidx, sidx, al, pidx, acc, den_sh, sem):
        c = lax.axis_index("c")
        s = lax.axis_index("s")
        base_e = s * PER_TILE

        pltpu.sync_copy(srcg.at[pl.ds(base_e, PER_TILE)], srcb)
        pltpu.sync_copy(dstg.at[pl.ds(base_e, PER_TILE)], dstb)

        z16f = jnp.zeros((16,), jnp.float32)
        lanes = lax.iota(jnp.int32, 16)

        def _zrows(r, _):
            for kk in range(C // 16):
                rows[r, pl.ds(kk * 16, 16)] = z16f
            return 0
        lax.fori_loop(0, KCH, _zrows, 0)

        def _pini(i, _):
            pidx[pl.ds(i * 16, 16)] = lanes + i * 16
            return 0
        lax.fori_loop(0, NR // 16, _pini, 0)

        for g in range(2):
            hh = 2 * g + c

            pltpu.sync_copy(asad.at[hh, 0], tas)
            pltpu.sync_copy(asad.at[hh, 1], tad)

            def _zden(r, _):
                for kk in range(8):
                    den[r, pl.ds(kk * 16, 16)] = z16f
                return 0
            lax.fori_loop(0, NR, _zden, 0)
            for kk in range(NSL // KCH):
                pltpu.sync_copy(rows, acc.at[pl.ds(s * NSL + kk * KCH, KCH)])
            pltpu.sync_copy(den.at[pl.ds(0, NR // 16)],
                            den_sh.at[pl.ds(s * (NR // 16), NR // 16)])
            plsc.subcore_barrier()

            def _passA(gg, _):
                o = gg * 16
                sv = srcb[pl.ds(o, 16)]
                dv = dstb[pl.ds(o, 16)]
                dhi = lax.shift_right_logical(dv, 7)
                dlo = lax.bitwise_and(dv, 127)
                ev = plsc.load_gather(tas, [sv]) + plsc.load_gather(tad, [dv])
                ev = jnp.where(ev > 0, ev, 0.2 * ev)
                plsc.addupdate_scatter(den, [dhi, dlo], jnp.exp(ev))
                return 0
            if _PASSA:
                lax.fori_loop(0, PER_TILE // 16, _passA, 0)
                pltpu.sync_copy(den, den_sh.at[pidx], add=True)
            plsc.subcore_barrier()
            pltpu.sync_copy(den_sh, den)

            def _chunk(ch, _):
                eb = ch * KCH

                def _idx(gg, _):
                    o = eb + gg * 16
                    gidx[pl.ds(gg * 16, 16)] = srcb[pl.ds(o, 16)] * H + hh
                    sidx[pl.ds(gg * 16, 16)] = dstb[pl.ds(o, 16)]
                    return 0
                lax.fori_loop(0, KCH // 16, _idx, 0)

                cp = pltpu.async_copy(ht.at[gidx], rows, sem)

                def _alpha(gg, _):
                    o = eb + gg * 16
                    sv = srcb[pl.ds(o, 16)]
                    dv = dstb[pl.ds(o, 16)]
                    dhi = lax.shift_right_logical(dv, 7)
                    dlo = lax.bitwise_and(dv, 127)
                    ev = plsc.load_gather(tas, [sv]) + plsc.load_gather(tad, [dv])
                    ev = jnp.where(ev > 0, ev, 0.2 * ev)
                    dd = plsc.load_gather(den, [dhi, dlo])
                    al[pl.ds(gg * 16, 16)] = jnp.exp(ev) / (dd + 1e-16)
                    return 0
                lax.fori_loop(0, KCH // 16, _alpha, 0)

                cp.wait()

                def _scale(gg, _):
                    av = al[pl.ds(gg * 16, 16)]
                    for l in range(16):
                        r = gg * 16 + l
                        a0 = av[l]
                        for kk in range(C // 16):
                            rows[r, pl.ds(kk * 16, 16)] = (
                                rows[r, pl.ds(kk * 16, 16)] * a0)
                    return 0
                lax.fori_loop(0, KCH // 16, _scale, 0)

                pltpu.sync_copy(rows, acc.at[sidx], add=True)
                return 0
            if _PASSB:
                lax.fori_loop(0, NCH, _chunk, 0)

            plsc.subcore_barrier()
            pltpu.sync_copy(acc.at[pl.ds(s * NSL, NSL)],
                            out.at[g, c, pl.ds(s * NSL, NSL)])
            if g == 0:
                def _zrows2(r, _):
                    for kk in range(C // 16):
                        rows[r, pl.ds(kk * 16, 16)] = z16f
                    return 0
                lax.fori_loop(0, KCH, _zrows2, 0)

    return k


_sc_edge_l1 = _make_sc_edge_kernel(C1)
_sc_edge_l2 = _make_sc_edge_kernel(C2)


def _attn_mats(a_s, a_d, ch):
    eyeH = jnp.eye(H, dtype=jnp.float32)
    As = (a_s[:, :, None] * eyeH[:, None, :]).reshape(H * ch, H)
    Ad = (a_d[:, :, None] * eyeH[:, None, :]).reshape(H * ch, H)
    return jnp.concatenate([As, Ad], axis=1)


_PERM = (0, 4, 1, 5, 2, 6, 3, 7)


def kernel(x, edge_index, W1, a_src1, a_dst1, b1, g1, be1, rm1, rv1,
           W2, a_src2, a_dst2, b2, g2, be2, rm2, rv2, Wfc, bfc):
    loops = jnp.arange(N, dtype=jnp.int32)
    padi = jnp.full((EPAD - E - N,), N, jnp.int32)
    src = jnp.concatenate([edge_index[0], loops, padi])
    dst = jnp.concatenate([edge_index[1], loops, padi])

    xp = jnp.zeros((NPAD, D), jnp.float32).at[:N].set(x)

    ht1, aa1 = _tc_layer1(xp, W1, _attn_mats(a_src1, a_dst1, C1))
    asad1 = aa1.T[jnp.array(_PERM)].reshape(H, 2, NPAD)
    acc1 = _sc_edge_l1(ht1.reshape(H * NPAD, C1), src, dst, asad1)

    ht2, aa2 = _tc_layer2(acc1, b1.reshape(1, -1), g1.reshape(1, -1),
                          be1.reshape(1, -1), rm1.reshape(1, -1),
                          rv1.reshape(1, -1), W2,
                          _attn_mats(a_src2, a_dst2, C2))
    asad2 = aa2.T[jnp.array(_PERM)].reshape(H, 2, NPAD)
    acc2 = _sc_edge_l2(ht2.reshape(H * NPAD, C2), src, dst, asad2)

    y = _tc_final(acc2, b2.reshape(1, -1), g2.reshape(1, -1),
                  be2.reshape(1, -1), rm2.reshape(1, -1),
                  rv2.reshape(1, -1), Wfc, bfc.reshape(1, 1))
    return y[:N]
